# baseline (speedup 1.0000x reference)
import functools
import numpy as np
import jax
import jax.numpy as jnp
from jax import lax
from jax.experimental import pallas as pl
from jax.experimental.pallas import tpu as pltpu
from jax.experimental.pallas import tpu_sc as plsc

N = 10000
E = 160004
G = 64
H = 256
F_IN = 512

NPAD = 10240
NROWS_SUB = NPAD // 16
CHUNK = 128
EPS_ROW = 10240
EPS_DEG = 5120
EPAD = 163840
MPAD = 10240

_mesh = plsc.VectorSubcoreMesh(core_axis_name="c", subcore_axis_name="s")


def _matmul_kern(x_ref, w_ref, o_ref):
    o_ref[...] = jnp.dot(x_ref[...], w_ref[...], preferred_element_type=jnp.float32)


def _matmul(x, w, bm=1024):
    m, k = x.shape
    _, n = w.shape
    return pl.pallas_call(
        _matmul_kern,
        grid=(m // bm,),
        in_specs=[
            pl.BlockSpec((bm, k), lambda i: (i, 0)),
            pl.BlockSpec((k, n), lambda i: (0, 0)),
        ],
        out_specs=pl.BlockSpec((bm, n), lambda i: (i, 0)),
        out_shape=jax.ShapeDtypeStruct((m, n), jnp.float32),
    )(x, w)


NCH = EPS_ROW // CHUNK


NCH = EPS_ROW // CHUNK
BCH = 16


@functools.partial(
    pl.kernel,
    out_type=jax.ShapeDtypeStruct((2 * NPAD, 128), jnp.float32),
    mesh=_mesh,
    scratch_types=[
        pltpu.VMEM((BCH, CHUNK), jnp.int32),
        pltpu.VMEM((BCH, CHUNK), jnp.int32),
        pltpu.VMEM((2, CHUNK, 128), jnp.float32),
        pltpu.VMEM_SHARED((NPAD, 128), jnp.float32),
        pltpu.SemaphoreType.DMA,
        pltpu.SemaphoreType.DMA,
    ],
    compiler_params=pltpu.CompilerParams(needs_layout_passes=False),
)
def _edge_agg(hs_hbm, src0_hbm, src1_hbm, dst_hbm, zer2_hbm, out_hbm,
              src_v, dst_v, rows_v, acc_sh, sem0, sem1):
    c = lax.axis_index("c")
    s = lax.axis_index("s")
    pltpu.sync_copy(zer2_hbm.at[pl.ds(s * NROWS_SUB, NROWS_SUB)],
                    acc_sh.at[pl.ds(s * NROWS_SUB, NROWS_SUB)])
    plsc.subcore_barrier()

    sems = (sem0, sem1)

    def blkbody(blk, carry):
        cb = blk * BCH

        @pl.when(c == 0)
        def _():
            pltpu.sync_copy(src0_hbm.at[s, pl.ds(cb, BCH)], src_v)

        @pl.when(c == 1)
        def _():
            pltpu.sync_copy(src1_hbm.at[s, pl.ds(cb, BCH)], src_v)

        pltpu.sync_copy(dst_hbm.at[s, pl.ds(cb, BCH)], dst_v)

        for b in range(2):
            pltpu.async_copy(hs_hbm.at[src_v.at[b]], rows_v.at[b], sems[b])

        def body(step, carry2):
            for b in range(2):
                i = step * 2 + b
                pltpu.make_async_copy(hs_hbm.at[src_v.at[i]], rows_v.at[b], sems[b]).wait()
                pltpu.sync_copy(rows_v.at[b], acc_sh.at[dst_v.at[i]], add=True)

                @pl.when(i + 2 < BCH)
                def _():
                    pltpu.async_copy(hs_hbm.at[src_v.at[i + 2]], rows_v.at[b], sems[b])

            return carry2

        lax.fori_loop(0, BCH // 2, body, 0)
        return carry

    lax.fori_loop(0, NCH // BCH, blkbody, 0)
    plsc.subcore_barrier()
    pltpu.sync_copy(acc_sh.at[pl.ds(s * NROWS_SUB, NROWS_SUB)],
                    out_hbm.at[pl.ds(c * NPAD + s * NROWS_SUB, NROWS_SUB)])


@functools.partial(
    pl.kernel,
    out_type=jax.ShapeDtypeStruct((2 * NPAD,), jnp.float32),
    mesh=_mesh,
    scratch_types=[
        pltpu.VMEM((NPAD,), jnp.float32),
        pltpu.VMEM((CHUNK,), jnp.int32),
        pltpu.VMEM((CHUNK,), jnp.int32),
        pltpu.VMEM((CHUNK,), jnp.float32),
        pltpu.VMEM((NROWS_SUB,), jnp.float32),
        pltpu.VMEM_SHARED((NPAD,), jnp.float32),
    ],
    compiler_params=pltpu.CompilerParams(needs_layout_passes=False),
)
def _deg_scatter(kp_hbm, src_hbm, dst_hbm, out_hbm,
                 kp_v, src_v, dst_v, vals_v, bnc_v, deg_sh):
    c = lax.axis_index("c")
    s = lax.axis_index("s")

    def zbody(j, carry):
        bnc_v[pl.ds(j * 16, 16)] = jnp.zeros((16,), jnp.float32)
        return carry

    lax.fori_loop(0, NROWS_SUB // 16, zbody, 0)
    pltpu.sync_copy(bnc_v, deg_sh.at[pl.ds(s * NROWS_SUB, NROWS_SUB)])
    pltpu.sync_copy(kp_hbm, kp_v)
    plsc.subcore_barrier()

    w = s * 2 + c
    base0 = w * EPS_DEG

    def body(i, carry):
        base = base0 + i * CHUNK
        pltpu.sync_copy(src_hbm.at[pl.ds(base, CHUNK)], src_v)
        pltpu.sync_copy(dst_hbm.at[pl.ds(base, CHUNK)], dst_v)
        for k in range(CHUNK // 16):
            sl = pl.ds(k * 16, 16)
            vals_v[sl] = plsc.load_gather(kp_v, [src_v[sl]])
        pltpu.sync_copy(vals_v, deg_sh.at[dst_v], add=True)
        return carry

    lax.fori_loop(0, EPS_DEG // CHUNK, body, 0)
    plsc.subcore_barrier()
    pltpu.sync_copy(deg_sh.at[pl.ds(s * NROWS_SUB, NROWS_SUB)], bnc_v)
    pltpu.sync_copy(bnc_v, out_hbm.at[pl.ds(c * NPAD + s * NROWS_SUB, NROWS_SUB)])


def _readout_kern(starts_ref, x_ref, kp_ref, o_ref):
    g = pl.program_id(0)
    start = starts_ref[g]
    end = starts_ref[g + 1]
    base = (start // 8) * 8
    nblk = (end - base + 7) // 8

    def body(j, carry):
        summ, cnt, mx = carry
        row0 = pl.multiple_of(base + j * 8, 8)
        blk = x_ref[pl.ds(row0, 8), :]
        kpb = kp_ref[pl.ds(row0, 8), :]
        rid = row0 + lax.broadcasted_iota(jnp.int32, (8, 1), 0)
        inseg = (rid >= start) & (rid < end)
        summ = summ + jnp.where(inseg, blk, 0.0)
        cnt = cnt + jnp.where(inseg, kpb, 0.0)
        mx = jnp.maximum(mx, jnp.where(inseg & (kpb > 0.0), blk, -jnp.inf))
        return summ, cnt, mx

    init = (
        jnp.zeros((8, H), jnp.float32),
        jnp.zeros((8, 1), jnp.float32),
        jnp.full((8, H), -jnp.inf, jnp.float32),
    )
    summ, cnt, mx = lax.fori_loop(0, nblk, body, init)
    s1 = jnp.sum(summ, axis=0, keepdims=True)
    c1 = jnp.sum(cnt)
    m1 = jnp.max(mx, axis=0, keepdims=True)
    mean = s1 / jnp.maximum(c1, 1.0)
    m1 = jnp.where(jnp.isfinite(m1), m1, 0.0)
    o_ref[...] = jnp.concatenate([m1, mean], axis=1)[None]


def _readout(x, kp, starts):
    xp = jnp.pad(x, ((0, MPAD - N), (0, 0)))
    kpp = jnp.pad(kp, (0, MPAD - N))[:, None]
    return pl.pallas_call(
        _readout_kern,
        grid=(G,),
        in_specs=[
            pl.BlockSpec(memory_space=pltpu.SMEM),
            pl.BlockSpec((MPAD, H), lambda g: (0, 0)),
            pl.BlockSpec((MPAD, 1), lambda g: (0, 0)),
        ],
        out_specs=pl.BlockSpec((1, 1, 2 * H), lambda g: (g, 0, 0)),
        out_shape=jax.ShapeDtypeStruct((G, 1, 2 * H), jnp.float32),
    )(starts, xp, kpp)[:, 0, :]


def kernel(data_x, data_edge_index, data_batch, W1, b1, W2, b2, W3, b3, edge_w, att1, att2):
    n = N
    src = jnp.pad(data_edge_index[0], (0, EPAD - E), constant_values=n)
    dst = jnp.pad(data_edge_index[1], (0, EPAD - E), constant_values=n)
    src3 = src.reshape(16, NCH, CHUNK)
    src3b = src3 + NPAD
    dst3 = dst.reshape(16, NCH, CHUNK)
    zer2 = jnp.zeros((NPAD, 128), jnp.float32)

    def gcn_masked(x, kp, W, b):
        kpp = jnp.pad(kp, (0, NPAD - n))
        deg2 = _deg_scatter(kpp, src, dst)
        deg0 = (deg2[:NPAD] + deg2[NPAD:])[:n]
        d = kp * lax.rsqrt(deg0 + 1.0)
        h = _matmul(jnp.pad(x, ((0, MPAD - n), (0, 0))), W)[:n]
        hs = d[:, None] * h
        hsp = jnp.pad(hs, ((0, NPAD - n), (0, 0)))
        hs2 = jnp.concatenate([hsp[:, :128], hsp[:, 128:]], axis=0)
        agg2 = _edge_agg(hs2, src3, src3b, dst3, zer2)
        agg0 = jnp.concatenate([agg2[:NPAD][:n], agg2[NPAD:][:n]], axis=1)
        out = d[:, None] * agg0 + (kp / (deg0 + 1.0))[:, None] * h
        return jax.nn.relu(out + b) * kp[:, None]

    def pool_masked(x, kp, att, k):
        score = (x @ att) / (jnp.linalg.norm(att) + 1e-8)
        score_m = jnp.where(kp > 0, score, -jnp.inf)
        kth = jax.lax.top_k(score_m, k)[0][-1]
        gt = score_m > kth
        eq = score_m == kth
        r = k - jnp.sum(gt)
        tie_rank = jnp.cumsum(eq.astype(jnp.int32))
        keep = (gt | (eq & (tie_rank <= r))).astype(x.dtype)
        xk = x * keep[:, None] * jnp.tanh(score)[:, None]
        return xk, keep

    starts = jnp.searchsorted(data_batch, jnp.arange(G + 1, dtype=jnp.int32)).astype(jnp.int32)

    def readout_masked(x, kp):
        return _readout(x, kp, starts)

    ones = jnp.ones((n,), jnp.float32)
    x = gcn_masked(data_x, ones, W1, b1)
    x, kp1 = pool_masked(x, ones, att1, int(np.ceil(n * 0.5)))
    x1 = readout_masked(x, kp1)
    x = gcn_masked(x, kp1, W2, b2)
    x, kp2 = pool_masked(x, kp1, att2, int(np.ceil(n * 0.5 * 0.5)))
    x2 = readout_masked(x, kp2)
    x = gcn_masked(x, kp2, W3, b3)
    x3 = readout_masked(x, kp2)
    return jax.nn.relu(x1) + jax.nn.relu(x2) + jax.nn.relu(x3)


# speedup vs baseline: 1.0299x; 1.0299x over previous
import functools
import numpy as np
import jax
import jax.numpy as jnp
from jax import lax
from jax.experimental import pallas as pl
from jax.experimental.pallas import tpu as pltpu
from jax.experimental.pallas import tpu_sc as plsc

N = 10000
E = 160004
G = 64
H = 256
F_IN = 512

NPAD = 10240
NROWS_SUB = NPAD // 16
CHUNK = 128
EPS_ROW = 10240
EPS_DEG = 5120
EPAD = 163840
MPAD = 10240

_mesh = plsc.VectorSubcoreMesh(core_axis_name="c", subcore_axis_name="s")


def _matmul_kern(x_ref, w_ref, o_ref):
    o_ref[...] = jnp.dot(x_ref[...], w_ref[...], preferred_element_type=jnp.float32)


def _matmul(x, w, bm=1024):
    m, k = x.shape
    _, n = w.shape
    return pl.pallas_call(
        _matmul_kern,
        grid=(m // bm,),
        in_specs=[
            pl.BlockSpec((bm, k), lambda i: (i, 0)),
            pl.BlockSpec((k, n), lambda i: (0, 0)),
        ],
        out_specs=pl.BlockSpec((bm, n), lambda i: (i, 0)),
        out_shape=jax.ShapeDtypeStruct((m, n), jnp.float32),
    )(x, w)


NCH = EPS_ROW // CHUNK
BCH = 16


@functools.partial(
    pl.kernel,
    out_type=jax.ShapeDtypeStruct((2 * NPAD, 128), jnp.float32),
    mesh=_mesh,
    scratch_types=[
        pltpu.VMEM((BCH, CHUNK), jnp.int32),
        pltpu.VMEM((BCH, CHUNK), jnp.int32),
        pltpu.VMEM((2, CHUNK, 128), jnp.float32),
        pltpu.VMEM_SHARED((NPAD, 128), jnp.float32),
        pltpu.SemaphoreType.DMA,
        pltpu.SemaphoreType.DMA,
    ],
    compiler_params=pltpu.CompilerParams(needs_layout_passes=False),
)
def _edge_agg(hs_hbm, src_hbm, dst_hbm, zer2_hbm, out_hbm,
              src_v, dst_v, rows_v, acc_sh, sem0, sem1):
    c = lax.axis_index("c")
    s = lax.axis_index("s")
    pltpu.sync_copy(zer2_hbm.at[pl.ds(s * NROWS_SUB, NROWS_SUB)],
                    acc_sh.at[pl.ds(s * NROWS_SUB, NROWS_SUB)])
    plsc.subcore_barrier()

    off = c * NPAD
    sems = (sem0, sem1)

    def blkbody(blk, carry):
        cb = blk * BCH
        pltpu.sync_copy(src_hbm.at[s, pl.ds(cb, BCH)], src_v)
        pltpu.sync_copy(dst_hbm.at[s, pl.ds(cb, BCH)], dst_v)

        def ibody(r, carry2):
            for k in range(CHUNK // 16):
                sl = pl.ds(k * 16, 16)
                src_v[r, sl] = src_v[r, sl] + off
            return carry2

        lax.fori_loop(0, BCH, ibody, 0)

        for b in range(2):
            pltpu.async_copy(hs_hbm.at[src_v.at[b]], rows_v.at[b], sems[b])

        def body(step, carry2):
            for b in range(2):
                i = step * 2 + b
                pltpu.make_async_copy(hs_hbm.at[src_v.at[i]], rows_v.at[b], sems[b]).wait()
                pltpu.sync_copy(rows_v.at[b], acc_sh.at[dst_v.at[i]], add=True)

                @pl.when(i + 2 < BCH)
                def _():
                    pltpu.async_copy(hs_hbm.at[src_v.at[i + 2]], rows_v.at[b], sems[b])

            return carry2

        lax.fori_loop(0, BCH // 2, body, 0)
        return carry

    lax.fori_loop(0, NCH // BCH, blkbody, 0)
    plsc.subcore_barrier()
    pltpu.sync_copy(acc_sh.at[pl.ds(s * NROWS_SUB, NROWS_SUB)],
                    out_hbm.at[pl.ds(c * NPAD + s * NROWS_SUB, NROWS_SUB)])


@functools.partial(
    pl.kernel,
    out_type=jax.ShapeDtypeStruct((2 * NPAD,), jnp.float32),
    mesh=_mesh,
    scratch_types=[
        pltpu.VMEM((NPAD,), jnp.float32),
        pltpu.VMEM((CHUNK,), jnp.int32),
        pltpu.VMEM((CHUNK,), jnp.int32),
        pltpu.VMEM((CHUNK,), jnp.float32),
        pltpu.VMEM((NROWS_SUB,), jnp.float32),
        pltpu.VMEM_SHARED((NPAD,), jnp.float32),
    ],
    compiler_params=pltpu.CompilerParams(needs_layout_passes=False),
)
def _deg_scatter(kp_hbm, src_hbm, dst_hbm, out_hbm,
                 kp_v, src_v, dst_v, vals_v, bnc_v, deg_sh):
    c = lax.axis_index("c")
    s = lax.axis_index("s")

    def zbody(j, carry):
        bnc_v[pl.ds(j * 16, 16)] = jnp.zeros((16,), jnp.float32)
        return carry

    lax.fori_loop(0, NROWS_SUB // 16, zbody, 0)
    pltpu.sync_copy(bnc_v, deg_sh.at[pl.ds(s * NROWS_SUB, NROWS_SUB)])
    pltpu.sync_copy(kp_hbm, kp_v)
    plsc.subcore_barrier()

    w = s * 2 + c
    base0 = w * EPS_DEG

    def body(i, carry):
        base = base0 + i * CHUNK
        pltpu.sync_copy(src_hbm.at[pl.ds(base, CHUNK)], src_v)
        pltpu.sync_copy(dst_hbm.at[pl.ds(base, CHUNK)], dst_v)
        for k in range(CHUNK // 16):
            sl = pl.ds(k * 16, 16)
            vals_v[sl] = plsc.load_gather(kp_v, [src_v[sl]])
        pltpu.sync_copy(vals_v, deg_sh.at[dst_v], add=True)
        return carry

    lax.fori_loop(0, EPS_DEG // CHUNK, body, 0)
    plsc.subcore_barrier()
    pltpu.sync_copy(deg_sh.at[pl.ds(s * NROWS_SUB, NROWS_SUB)], bnc_v)
    pltpu.sync_copy(bnc_v, out_hbm.at[pl.ds(c * NPAD + s * NROWS_SUB, NROWS_SUB)])


def _readout_kern(starts_ref, x_ref, kp_ref, o_ref):
    g = pl.program_id(0)
    start = starts_ref[g]
    end = starts_ref[g + 1]
    base = (start // 8) * 8
    nblk = (end - base + 7) // 8

    def body(j, carry):
        summ, cnt, mx = carry
        row0 = pl.multiple_of(base + j * 8, 8)
        blk = x_ref[pl.ds(row0, 8), :]
        kpb = kp_ref[pl.ds(row0, 8), :]
        rid = row0 + lax.broadcasted_iota(jnp.int32, (8, 1), 0)
        inseg = (rid >= start) & (rid < end)
        summ = summ + jnp.where(inseg, blk, 0.0)
        cnt = cnt + jnp.where(inseg, kpb, 0.0)
        mx = jnp.maximum(mx, jnp.where(inseg & (kpb > 0.0), blk, -jnp.inf))
        return summ, cnt, mx

    init = (
        jnp.zeros((8, H), jnp.float32),
        jnp.zeros((8, 1), jnp.float32),
        jnp.full((8, H), -jnp.inf, jnp.float32),
    )
    summ, cnt, mx = lax.fori_loop(0, nblk, body, init)
    s1 = jnp.sum(summ, axis=0, keepdims=True)
    c1 = jnp.sum(cnt)
    m1 = jnp.max(mx, axis=0, keepdims=True)
    mean = s1 / jnp.maximum(c1, 1.0)
    m1 = jnp.where(jnp.isfinite(m1), m1, 0.0)
    o_ref[...] = jnp.concatenate([m1, mean], axis=1)[None]


def _readout(x, kp, starts):
    xp = jnp.pad(x, ((0, MPAD - N), (0, 0)))
    kpp = jnp.pad(kp, (0, MPAD - N))[:, None]
    return pl.pallas_call(
        _readout_kern,
        grid=(G,),
        in_specs=[
            pl.BlockSpec(memory_space=pltpu.SMEM),
            pl.BlockSpec((MPAD, H), lambda g: (0, 0)),
            pl.BlockSpec((MPAD, 1), lambda g: (0, 0)),
        ],
        out_specs=pl.BlockSpec((1, 1, 2 * H), lambda g: (g, 0, 0)),
        out_shape=jax.ShapeDtypeStruct((G, 1, 2 * H), jnp.float32),
    )(starts, xp, kpp)[:, 0, :]


def kernel(data_x, data_edge_index, data_batch, W1, b1, W2, b2, W3, b3, edge_w, att1, att2):
    n = N
    src = jnp.pad(data_edge_index[0], (0, EPAD - E), constant_values=n)
    dst = jnp.pad(data_edge_index[1], (0, EPAD - E), constant_values=n)
    src3 = src.reshape(16, NCH, CHUNK)
    dst3 = dst.reshape(16, NCH, CHUNK)
    zer2 = jnp.zeros((NPAD, 128), jnp.float32)

    def gcn_masked(x, kp, W, b):
        kpp = jnp.pad(kp, (0, NPAD - n))
        deg2 = _deg_scatter(kpp, src, dst)
        deg0 = (deg2[:NPAD] + deg2[NPAD:])[:n]
        d = kp * lax.rsqrt(deg0 + 1.0)
        h = _matmul(jnp.pad(x, ((0, MPAD - n), (0, 0))), W)[:n]
        hs = d[:, None] * h
        hsp = jnp.pad(hs, ((0, NPAD - n), (0, 0)))
        hs2 = jnp.concatenate([hsp[:, :128], hsp[:, 128:]], axis=0)
        agg2 = _edge_agg(hs2, src3, dst3, zer2)
        agg0 = jnp.concatenate([agg2[:NPAD][:n], agg2[NPAD:][:n]], axis=1)
        out = d[:, None] * agg0 + (kp / (deg0 + 1.0))[:, None] * h
        return jax.nn.relu(out + b) * kp[:, None]

    def pool_masked(x, kp, att, k):
        score = (x @ att) / (jnp.linalg.norm(att) + 1e-8)
        score_m = jnp.where(kp > 0, score, -jnp.inf)
        kth = jax.lax.top_k(score_m, k)[0][-1]
        gt = score_m > kth
        eq = score_m == kth
        r = k - jnp.sum(gt)
        tie_rank = jnp.cumsum(eq.astype(jnp.int32))
        keep = (gt | (eq & (tie_rank <= r))).astype(x.dtype)
        xk = x * keep[:, None] * jnp.tanh(score)[:, None]
        return xk, keep

    starts = jnp.searchsorted(data_batch, jnp.arange(G + 1, dtype=jnp.int32)).astype(jnp.int32)

    def readout_masked(x, kp):
        return _readout(x, kp, starts)

    ones = jnp.ones((n,), jnp.float32)
    x = gcn_masked(data_x, ones, W1, b1)
    x, kp1 = pool_masked(x, ones, att1, int(np.ceil(n * 0.5)))
    x1 = readout_masked(x, kp1)
    x = gcn_masked(x, kp1, W2, b2)
    x, kp2 = pool_masked(x, kp1, att2, int(np.ceil(n * 0.5 * 0.5)))
    x2 = readout_masked(x, kp2)
    x = gcn_masked(x, kp2, W3, b3)
    x3 = readout_masked(x, kp2)
    return jax.nn.relu(x1) + jax.nn.relu(x2) + jax.nn.relu(x3)
